# Initial kernel scaffold; baseline (speedup 1.0000x reference)
import jax
import jax.numpy as jnp
from jax.experimental import pallas as pl


def kernel(boxes, scores):
    raise NotImplementedError("write your pallas kernel here")


# speedup vs baseline: 87.5055x; 87.5055x over previous
import functools

import jax
import jax.numpy as jnp
from jax import lax
from jax.experimental import pallas as pl
from jax.experimental.pallas import tpu as pltpu

_THRESH_PROB = 0.7
_THRESH_IOU = 0.7

_P = 5120
_RB = 256
_CB = 512
_NBI = _P // _RB
_NBJ = _P // _CB


def _nms_body(x1r, y1r, x2r, y2r, sr, x1c, y1c, x2c, y2c, sc, out_ref, kcol):
    valid0 = (sr[...] > _THRESH_PROB).astype(jnp.float32)
    kcol[...] = valid0

    def row_step(ib, changed):
        i0 = ib * _RB
        xi1 = x1r[pl.ds(i0, _RB), :]
        yi1 = y1r[pl.ds(i0, _RB), :]
        xi2 = x2r[pl.ds(i0, _RB), :]
        yi2 = y2r[pl.ds(i0, _RB), :]
        si = sr[pl.ds(i0, _RB), :]
        areai = (xi2 - xi1) * (yi2 - yi1)
        iidx = i0 + lax.broadcasted_iota(jnp.int32, (_RB, 1), 0)

        def col_step(jb, acc):
            j0 = jb * _CB
            xj1 = x1c[:, pl.ds(j0, _CB)]
            yj1 = y1c[:, pl.ds(j0, _CB)]
            xj2 = x2c[:, pl.ds(j0, _CB)]
            yj2 = y2c[:, pl.ds(j0, _CB)]
            sj = sc[:, pl.ds(j0, _CB)]
            areaj = (xj2 - xj1) * (yj2 - yj1)
            jidx = j0 + lax.broadcasted_iota(jnp.int32, (1, _CB), 1)
            iw = jnp.maximum(jnp.minimum(xi2, xj2) - jnp.maximum(xi1, xj1), 0.0)
            ih = jnp.maximum(jnp.minimum(yi2, yj2) - jnp.maximum(yi1, yj1), 0.0)
            inter = iw * ih
            union = areai + areaj - inter
            prec = (sj > si) | ((sj == si) & (jidx < iidx))
            m = (inter > _THRESH_IOU * (union + 1e-9)) & prec
            kj = kcol[pl.ds(j0, _CB), :]
            acc = acc + jnp.dot(m.astype(jnp.bfloat16), kj.astype(jnp.bfloat16),
                                preferred_element_type=jnp.float32)
            return acc

        acc = lax.fori_loop(0, _NBJ, col_step, jnp.zeros((_RB, 1), jnp.float32))
        newk = jnp.where((si > _THRESH_PROB) & (acc < 0.5), 1.0, 0.0)
        oldk = kcol[pl.ds(i0, _RB), :]
        kcol[pl.ds(i0, _RB), :] = newk
        return changed | jnp.any(newk != oldk)

    def sweep(carry):
        return (lax.fori_loop(0, _NBI, row_step, jnp.bool_(False)),)

    lax.while_loop(lambda c: c[0], sweep, (jnp.bool_(True),))

    k = kcol[...]
    z = jnp.zeros_like(k)
    out_ref[...] = jnp.concatenate(
        [x1r[...] * k, y1r[...] * k, x2r[...] * k, y2r[...] * k,
         sr[...] * k, z, z, z], axis=1)


@jax.jit
def kernel(boxes, scores):
    n = boxes.shape[0]
    pb = jnp.zeros((_P, 4), boxes.dtype).at[:n].set(boxes)
    ps = jnp.full((_P,), -1.0, scores.dtype).at[:n].set(scores)
    cols_r = [pb[:, i].reshape(_P, 1) for i in range(4)] + [ps.reshape(_P, 1)]
    cols_c = [pb[:, i].reshape(1, _P) for i in range(4)] + [ps.reshape(1, _P)]
    out8 = pl.pallas_call(
        _nms_body,
        out_shape=jax.ShapeDtypeStruct((_P, 8), jnp.float32),
        scratch_shapes=[pltpu.VMEM((_P, 1), jnp.float32)],
    )(*cols_r, *cols_c)
    return out8[:n, :5]
